# baseline (speedup 1.0000x reference)
import jax
import jax.numpy as jnp
from jax.experimental import pallas as pl
from jax.experimental.pallas import tpu as pltpu

E = 64
R = 8
D = 2048
DOUT = 2048
ALPHA = 16.0
EPS = 1e-06

BT = 256
NJ = 4
BD = DOUT // NJ
N_TOK = 4096


def _fused_kernel(x_ref, w_ref, b_ref, a_ref, bt_ref, g_ref, o_ref, mid_ref):
    j = pl.program_id(0)
    i = pl.program_id(1)
    x = x_ref[...]

    @pl.when(j == 0)
    def _():
        xn = x / (jnp.sqrt(jnp.sum(x * x, axis=1, keepdims=True)) + EPS)
        g = g_ref[...]
        gn = g / (jnp.sqrt(jnp.sum(g * g, axis=1, keepdims=True)) + EPS)
        scores = jax.lax.dot_general(
            xn, gn, (((1,), (1,)), ((), ())),
            preferred_element_type=jnp.float32) * (1.0 / (D ** 0.5))
        eidx = jax.lax.broadcasted_iota(jnp.int32, (BT, E), 1)
        m1 = jnp.max(scores, axis=1, keepdims=True)
        idx1 = jnp.min(jnp.where(scores == m1, eidx, E), axis=1, keepdims=True)
        masked = jnp.where(eidx == idx1, -jnp.inf, scores)
        m2 = jnp.max(masked, axis=1, keepdims=True)
        idx2 = jnp.min(jnp.where(masked == m2, eidx, E), axis=1, keepdims=True)
        e2 = jnp.exp(m2 - m1)
        denom = 1.0 + e2
        w1 = 1.0 / denom
        w2 = e2 / denom
        mid = jax.lax.dot_general(
            x, a_ref[...], (((1,), (1,)), ((), ())),
            preferred_element_type=jnp.float32)
        lane_e = jax.lax.broadcasted_iota(jnp.int32, (BT, E * R), 1) // R
        scale = ALPHA / float(R)
        mask = (jnp.where(lane_e == idx1, w1, 0.0)
                + jnp.where(lane_e == idx2, w2, 0.0)) * scale
        mid_ref[pl.ds(i * BT, BT), :] = mid * mask

    base = jax.lax.dot_general(
        x, w_ref[...], (((1,), (1,)), ((), ())),
        preferred_element_type=jnp.float32)
    mid = mid_ref[pl.ds(i * BT, BT), :]
    delta = jnp.dot(mid, bt_ref[...],
                    preferred_element_type=jnp.float32)
    o_ref[...] = base + delta + b_ref[...]


@jax.jit
def kernel(x, W, b, A_all, B_all, gate_vecs):
    batch, seq, d = x.shape
    n = batch * seq
    x_flat = x.reshape(n, d)
    A_flat = A_all.reshape(E * R, D)
    B_flat = B_all.transpose(0, 2, 1).reshape(E * R, DOUT)
    b2 = b.reshape(1, DOUT)

    grid = (NJ, n // BT)
    out = pl.pallas_call(
        _fused_kernel,
        grid=grid,
        in_specs=[
            pl.BlockSpec((BT, D), lambda j, i: (i, 0)),
            pl.BlockSpec((BD, D), lambda j, i: (j, 0)),
            pl.BlockSpec((1, BD), lambda j, i: (0, j)),
            pl.BlockSpec((E * R, D), lambda j, i: (0, 0)),
            pl.BlockSpec((E * R, BD), lambda j, i: (0, j)),
            pl.BlockSpec((E, D), lambda j, i: (0, 0)),
        ],
        out_specs=pl.BlockSpec((BT, BD), lambda j, i: (i, j)),
        out_shape=jax.ShapeDtypeStruct((n, DOUT), jnp.float32),
        scratch_shapes=[pltpu.VMEM((N_TOK, E * R), jnp.float32)],
    )(x_flat, W, b2, A_flat, B_flat, gate_vecs)
    return out.reshape(batch, seq, DOUT)


# speedup vs baseline: 1.4856x; 1.4856x over previous
import jax
import jax.numpy as jnp
from jax.experimental import pallas as pl
from jax.experimental.pallas import tpu as pltpu

E = 64
R = 8
D = 2048
DOUT = 2048
ALPHA = 16.0
EPS = 1e-06

BT = 512
NJ = 2
BD = DOUT // NJ
N_TOK = 4096


def _fused_kernel(x_ref, w_ref, b_ref, a_ref, bt_ref, g_ref, o_ref, mid_ref):
    j = pl.program_id(0)
    i = pl.program_id(1)
    x = x_ref[...]

    @pl.when(j == 0)
    def _():
        xn = x / (jnp.sqrt(jnp.sum(x * x, axis=1, keepdims=True)) + EPS)
        g = g_ref[...]
        gn = g / (jnp.sqrt(jnp.sum(g * g, axis=1, keepdims=True)) + EPS)
        scores = jax.lax.dot_general(
            xn, gn, (((1,), (1,)), ((), ())),
            preferred_element_type=jnp.float32) * (1.0 / (D ** 0.5))
        eidx = jax.lax.broadcasted_iota(jnp.int32, (BT, E), 1)
        m1 = jnp.max(scores, axis=1, keepdims=True)
        idx1 = jnp.min(jnp.where(scores == m1, eidx, E), axis=1, keepdims=True)
        masked = jnp.where(eidx == idx1, -jnp.inf, scores)
        m2 = jnp.max(masked, axis=1, keepdims=True)
        idx2 = jnp.min(jnp.where(masked == m2, eidx, E), axis=1, keepdims=True)
        e2 = jnp.exp(m2 - m1)
        denom = 1.0 + e2
        w1 = 1.0 / denom
        w2 = e2 / denom
        mid = jax.lax.dot_general(
            x, a_ref[...], (((1,), (1,)), ((), ())),
            preferred_element_type=jnp.float32)
        lane_e = jax.lax.broadcasted_iota(jnp.int32, (BT, E * R), 1) // R
        scale = ALPHA / float(R)
        mask = (jnp.where(lane_e == idx1, w1, 0.0)
                + jnp.where(lane_e == idx2, w2, 0.0)) * scale
        mid_ref[pl.ds(i * BT, BT), :] = mid * mask

    base = jax.lax.dot_general(
        x, w_ref[...], (((1,), (1,)), ((), ())),
        preferred_element_type=jnp.float32)
    mid = mid_ref[pl.ds(i * BT, BT), :]
    delta = jnp.dot(mid, bt_ref[...],
                    preferred_element_type=jnp.float32)
    o_ref[...] = base + delta + b_ref[...]


@jax.jit
def kernel(x, W, b, A_all, B_all, gate_vecs):
    batch, seq, d = x.shape
    n = batch * seq
    x_flat = x.reshape(n, d)
    A_flat = A_all.reshape(E * R, D)
    B_flat = B_all.transpose(0, 2, 1).reshape(E * R, DOUT)
    b2 = b.reshape(1, DOUT)

    grid = (NJ, n // BT)
    out = pl.pallas_call(
        _fused_kernel,
        grid=grid,
        in_specs=[
            pl.BlockSpec((BT, D), lambda j, i: (i, 0)),
            pl.BlockSpec((BD, D), lambda j, i: (j, 0)),
            pl.BlockSpec((1, BD), lambda j, i: (0, j)),
            pl.BlockSpec((E * R, D), lambda j, i: (0, 0)),
            pl.BlockSpec((E * R, BD), lambda j, i: (0, j)),
            pl.BlockSpec((E, D), lambda j, i: (0, 0)),
        ],
        out_specs=pl.BlockSpec((BT, BD), lambda j, i: (i, j)),
        out_shape=jax.ShapeDtypeStruct((n, DOUT), jnp.float32),
        scratch_shapes=[pltpu.VMEM((N_TOK, E * R), jnp.float32)],
    )(x_flat, W, b2, A_flat, B_flat, gate_vecs)
    return out.reshape(batch, seq, DOUT)
